# Initial kernel scaffold; baseline (speedup 1.0000x reference)
import jax
import jax.numpy as jnp
from jax.experimental import pallas as pl


def kernel(x, embedding):
    raise NotImplementedError("write your pallas kernel here")


# speedup vs baseline: 3.4578x; 3.4578x over previous
import jax
import jax.numpy as jnp
from jax.experimental import pallas as pl
from jax.experimental.pallas import tpu as pltpu

_K = 1024
_D = 256
_T = 1024
_NB = 8
_N = _NB * _T
_BETA = 0.25


def _vq_body(x_ref, e_ref, zq_ref, misc_ref, counts_ref, loss_ref):
    b = pl.program_id(0)
    xb = x_ref[0]
    e = e_ref[...]

    esq = jnp.sum(e * e, axis=1, keepdims=True)
    zsq = jnp.sum(xb * xb, axis=0, keepdims=True)
    mm = jax.lax.dot_general(
        e, xb, (((1,), (0,)), ((), ())),
        precision=jax.lax.Precision.DEFAULT,
        preferred_element_type=jnp.float32)
    d = (zsq + esq) - 2.0 * mm

    vmin = jnp.min(d, axis=0, keepdims=True)
    kio = jax.lax.broadcasted_iota(jnp.int32, (_K, _T), 0)
    idx = jnp.min(jnp.where(d == vmin, kio, _K), axis=0, keepdims=True)
    onehot = jnp.where(kio == idx, 1.0, 0.0).astype(jnp.float32)

    zq = jax.lax.dot_general(
        e, onehot, (((0,), (0,)), ((), ())),
        precision=jax.lax.Precision.DEFAULT,
        preferred_element_type=jnp.float32)
    zq_ref[0] = xb + (zq - xb)

    pc = jnp.sum(onehot, axis=1, keepdims=True)
    ls = jnp.sum(vmin)

    @pl.when(b == 0)
    def _():
        counts_ref[...] = pc
        loss_ref[0, 0] = ls

    @pl.when(b > 0)
    def _():
        counts_ref[...] = counts_ref[...] + pc
        loss_ref[0, 0] = loss_ref[0, 0] + ls

    @pl.when(b == _NB - 1)
    def _():
        em = counts_ref[...] * (1.0 / _N)
        s = -jnp.sum(em * jnp.log(em + 1e-10))
        pv = jnp.exp(jnp.full((1, 128), s, jnp.float32))
        lt = loss_ref[0, 0] * ((1.0 + _BETA) / (_N * _D))
        lane = jax.lax.broadcasted_iota(jnp.int32, (1, 128), 1)
        misc_ref[...] = jnp.where(lane == 0, lt, pv)


def kernel(x, embedding):
    xr = jnp.reshape(x, (_NB, _D, _T))
    zq, misc = pl.pallas_call(
        _vq_body,
        grid=(_NB,),
        in_specs=[pl.BlockSpec((1, _D, _T), lambda i: (i, 0, 0)),
                  pl.BlockSpec((_K, _D), lambda i: (0, 0))],
        out_specs=[pl.BlockSpec((1, _D, _T), lambda i: (i, 0, 0)),
                   pl.BlockSpec((1, 128), lambda i: (0, 0))],
        out_shape=[jax.ShapeDtypeStruct((_NB, _D, _T), jnp.float32),
                   jax.ShapeDtypeStruct((1, 128), jnp.float32)],
        scratch_shapes=[pltpu.VMEM((_K, 1), jnp.float32),
                        pltpu.SMEM((1, 1), jnp.float32)],
    )(xr, embedding)
    z_q = jnp.reshape(zq, (_NB, _D, 32, 32))
    return (misc[0, 0], z_q, misc[0, 1])


# speedup vs baseline: 3.4959x; 1.0110x over previous
import jax
import jax.numpy as jnp
from jax.experimental import pallas as pl
from jax.experimental.pallas import tpu as pltpu

_K = 1024
_D = 256
_T = 1024
_NB = 8
_N = _NB * _T
_BETA = 0.25


def _vq_body(x_ref, e_ref, zq_ref, misc_ref, counts_ref, loss_ref):
    b = pl.program_id(0)
    xb = x_ref[0]
    e = e_ref[...]

    esq = jnp.sum(e * e, axis=1, keepdims=True)
    zsq = jnp.sum(xb * xb, axis=0, keepdims=True)
    mm2 = jax.lax.dot_general(
        e + e, xb, (((1,), (0,)), ((), ())),
        precision=jax.lax.Precision.DEFAULT,
        preferred_element_type=jnp.float32)
    d = (zsq + esq) - mm2

    vmin = jnp.min(d, axis=0, keepdims=True)
    kio = jax.lax.broadcasted_iota(jnp.int32, (_K, _T), 0)
    idx = jnp.min(jnp.where(d == vmin, kio, _K), axis=0, keepdims=True)
    onehot = jnp.where(kio == idx, 1.0, 0.0).astype(jnp.float32)

    zq = jax.lax.dot_general(
        e, onehot, (((0,), (0,)), ((), ())),
        precision=jax.lax.Precision.DEFAULT,
        preferred_element_type=jnp.float32)
    zq_ref[0] = xb + (zq - xb)

    pc = jnp.sum(onehot, axis=1, keepdims=True)
    ls = jnp.sum(vmin)

    @pl.when(b == 0)
    def _():
        counts_ref[...] = pc
        loss_ref[0, 0] = ls

    @pl.when(b > 0)
    def _():
        counts_ref[...] = counts_ref[...] + pc
        loss_ref[0, 0] = loss_ref[0, 0] + ls

    @pl.when(b == _NB - 1)
    def _():
        em = counts_ref[...] * (1.0 / _N)
        s = -jnp.sum(em * jnp.log(em + 1e-10))
        pv = jnp.exp(jnp.full((1, 128), s, jnp.float32))
        lt = loss_ref[0, 0] * ((1.0 + _BETA) / (_N * _D))
        lane = jax.lax.broadcasted_iota(jnp.int32, (1, 128), 1)
        misc_ref[...] = jnp.where(lane == 0, lt, pv)


def kernel(x, embedding):
    xr = jnp.reshape(x, (_NB, _D, _T))
    zq, misc = pl.pallas_call(
        _vq_body,
        grid=(_NB,),
        in_specs=[pl.BlockSpec((1, _D, _T), lambda i: (i, 0, 0)),
                  pl.BlockSpec((_K, _D), lambda i: (0, 0))],
        out_specs=[pl.BlockSpec((1, _D, _T), lambda i: (i, 0, 0)),
                   pl.BlockSpec((1, 128), lambda i: (0, 0))],
        out_shape=[jax.ShapeDtypeStruct((_NB, _D, _T), jnp.float32),
                   jax.ShapeDtypeStruct((1, 128), jnp.float32)],
        scratch_shapes=[pltpu.VMEM((_K, 1), jnp.float32),
                        pltpu.SMEM((1, 1), jnp.float32)],
    )(xr, embedding)
    z_q = jnp.reshape(zq, (_NB, _D, 32, 32))
    return (misc[0, 0], z_q, misc[0, 1])
